# speedup vs baseline: 1.9901x; 1.9901x over previous
import functools

import jax
import jax.numpy as jnp
import numpy as np
from jax import lax
from jax.experimental import pallas as pl
from jax.experimental.pallas import tpu as pltpu
from jax.experimental.pallas import tpu_sc as plsc

VOCAB = 100000
D_MODEL = 768
MAX_LEN = 4096
PADDING_IDX = 1

NUM_CORES = 2
NUM_SUBCORES = 16
NUM_WORKERS = NUM_CORES * NUM_SUBCORES
LANES = 16


def _pos_encoding(max_len, d_model):
    pos = np.arange(max_len, dtype=np.float32)[:, None]
    _2i = np.arange(0, d_model, step=2, dtype=np.float32)
    enc = np.zeros((max_len, d_model), dtype=np.float32)
    enc[:, 0::2] = np.sin(pos / np.power(10000.0, _2i / d_model))
    enc[:, 1::2] = np.cos(pos / np.power(10000.0, _2i / d_model))
    return enc


_POS_ENC = _pos_encoding(MAX_LEN, D_MODEL)


@functools.partial(jax.jit, static_argnums=(2, 3))
def _embed(x_flat, table, seq_len, n_flat):
    b_per_w = n_flat // NUM_WORKERS
    chunk = 64
    n_chunks = b_per_w // chunk
    vregs = D_MODEL // LANES

    enc = jnp.asarray(_POS_ENC[:seq_len])

    mesh = plsc.VectorSubcoreMesh(
        core_axis_name="c", subcore_axis_name="s")

    @functools.partial(
        pl.kernel,
        out_type=jax.ShapeDtypeStruct((n_flat, D_MODEL), jnp.float32),
        mesh=mesh,
        scratch_types=[
            pltpu.VMEM((b_per_w,), jnp.int32),
            pltpu.VMEM((chunk, D_MODEL), jnp.float32),
            pltpu.VMEM((chunk, D_MODEL), jnp.float32),
            pltpu.SemaphoreType.DMA,
        ],
    )
    def k(x_hbm, table_hbm, enc_hbm, out_hbm, idx_v, e_v, g_v, sem):
        wid = lax.axis_index("s") * NUM_CORES + lax.axis_index("c")
        base = wid * b_per_w
        s0 = lax.rem(base, seq_len)
        pltpu.sync_copy(x_hbm.at[pl.ds(base, b_per_w)], idx_v)
        for c in range(n_chunks):
            off = c * chunk
            cp = pltpu.async_copy(
                table_hbm.at[idx_v.at[pl.ds(off, chunk)]], g_v, sem)
            pltpu.sync_copy(enc_hbm.at[pl.ds(s0 + off, chunk)], e_v)
            cp.wait()
            for g in range(chunk // LANES):
                v = idx_v[pl.ds(off + g * LANES, LANES)]
                mf = jnp.where(v == PADDING_IDX, 0.0, 1.0)
                for r in range(LANES):
                    row = g * LANES + r
                    m = mf[r]

                    def jbody(j, _, row=row, m=m):
                        sl = pl.ds(j * LANES, LANES)
                        e_v[row, sl] = g_v[row, sl] * m + e_v[row, sl]
                        return 0

                    lax.fori_loop(0, vregs, jbody, 0)
            pltpu.sync_copy(e_v, out_hbm.at[pl.ds(base + off, chunk)])

    return k(x_flat, table, enc)


def kernel(x, table):
    b, s = x.shape
    out = _embed(x.reshape(b * s), table, s, b * s)
    return out.reshape(b, s, D_MODEL)
